# baseline (speedup 1.0000x reference)
import functools

import jax
import jax.numpy as jnp
from jax import lax
from jax.experimental import pallas as pl
from jax.experimental.pallas import tpu as pltpu
from jax.experimental.pallas import tpu_sc as plsc

_N_TD = 1000000
_BATCH = 16384

_info = plsc.get_sparse_core_info()
_NC, _NS = 1, _info.num_subcores
_NW = _NC * _NS
_B_PER_W = _BATCH // _NW

_mesh = plsc.VectorSubcoreMesh(
    core_axis_name="c", subcore_axis_name="s", num_cores=_NC
)


@functools.partial(
    pl.kernel,
    mesh=_mesh,
    out_type=jax.ShapeDtypeStruct((_BATCH,), jnp.float32),
    scratch_types=[
        pltpu.VMEM((_B_PER_W,), jnp.int32),
        pltpu.VMEM((_B_PER_W,), jnp.float32),
        pltpu.SemaphoreType.DMA,
        pltpu.SemaphoreType.DMA,
        pltpu.SemaphoreType.DMA,
        pltpu.SemaphoreType.DMA,
        pltpu.SemaphoreType.DMA,
        pltpu.SemaphoreType.DMA,
        pltpu.SemaphoreType.DMA,
        pltpu.SemaphoreType.DMA,
        pltpu.SemaphoreType.DMA,
    ],
)
def _gather_kernel(idx_hbm, table_hbm, out_hbm, idx_v, vals_v, *sems):
    wid = lax.axis_index("s") * _NC + lax.axis_index("c")
    base = wid * _B_PER_W
    nchunk = 4
    csz = _B_PER_W // nchunk
    si, sg = sems[:nchunk], sems[nchunk : 2 * nchunk]
    so = sems[2 * nchunk]
    ci = [
        pltpu.async_copy(
            idx_hbm.at[pl.ds(base + j * csz, csz)],
            idx_v.at[pl.ds(j * csz, csz)],
            si[j],
        )
        for j in range(nchunk)
    ]
    gs = []
    for j in range(nchunk):
        ci[j].wait()
        sl = pl.ds(j * csz, csz)
        gs.append(pltpu.async_copy(table_hbm.at[idx_v.at[sl]], vals_v.at[sl], sg[j]))
    os = []
    for j in range(nchunk):
        gs[j].wait()
        sl = pl.ds(j * csz, csz)
        os.append(
            pltpu.async_copy(vals_v.at[sl], out_hbm.at[pl.ds(base + j * csz, csz)], so)
        )
    for o in os:
        o.wait()


@jax.jit
def kernel(td_id, bias_weight):
    flat = _gather_kernel(td_id, bias_weight.reshape(_N_TD))
    return flat.reshape(_BATCH, 1)


# speedup vs baseline: 1.0019x; 1.0019x over previous
import functools

import jax
import jax.numpy as jnp
from jax import lax
from jax.experimental import pallas as pl
from jax.experimental.pallas import tpu as pltpu
from jax.experimental.pallas import tpu_sc as plsc

_N_TD = 1000000
_BATCH = 16384

_info = plsc.get_sparse_core_info()
_NC, _NS = 1, _info.num_subcores
_NW = _NC * _NS
_B_PER_W = _BATCH // _NW

_mesh = plsc.VectorSubcoreMesh(
    core_axis_name="c", subcore_axis_name="s", num_cores=_NC
)


@functools.partial(
    pl.kernel,
    mesh=_mesh,
    out_type=jax.ShapeDtypeStruct((_BATCH,), jnp.float32),
    scratch_types=[
        pltpu.VMEM((_B_PER_W,), jnp.int32),
        pltpu.VMEM((_B_PER_W,), jnp.float32),
        pltpu.SemaphoreType.DMA,
        pltpu.SemaphoreType.DMA,
        pltpu.SemaphoreType.DMA,
        pltpu.SemaphoreType.DMA,
        pltpu.SemaphoreType.DMA,
    ],
)
def _gather_kernel(
    idx_hbm, table_hbm, out_hbm, idx_v, vals_v, si0, si1, sg0, sg1, so
):
    wid = lax.axis_index("s") * _NC + lax.axis_index("c")
    base = wid * _B_PER_W
    half = _B_PER_W // 2
    lo, hi = pl.ds(0, half), pl.ds(half, half)
    ci0 = pltpu.async_copy(idx_hbm.at[pl.ds(base, half)], idx_v.at[lo], si0)
    ci1 = pltpu.async_copy(idx_hbm.at[pl.ds(base + half, half)], idx_v.at[hi], si1)
    ci0.wait()
    g0 = pltpu.async_copy(table_hbm.at[idx_v.at[lo]], vals_v.at[lo], sg0)
    ci1.wait()
    g1 = pltpu.async_copy(table_hbm.at[idx_v.at[hi]], vals_v.at[hi], sg1)
    g0.wait()
    o0 = pltpu.async_copy(vals_v.at[lo], out_hbm.at[pl.ds(base, half)], so)
    g1.wait()
    o1 = pltpu.async_copy(vals_v.at[hi], out_hbm.at[pl.ds(base + half, half)], so)
    o0.wait()
    o1.wait()


@jax.jit
def kernel(td_id, bias_weight):
    flat = _gather_kernel(td_id, bias_weight.reshape(_N_TD))
    return flat.reshape(_BATCH, 1)
